# baseline (speedup 1.0000x reference)
import jax
import jax.numpy as jnp
from jax.experimental import pallas as pl
from jax.experimental.pallas import tpu as pltpu


def _proj_kernel(x_ref,
                 w_ref,
                 b_ref,
                 out_ref):
    w = w_ref[...]
    b = b_ref[...]
    tcp = out_ref.shape[0]
    for t in range(tcp):
        g = jnp.dot(x_ref[:, t, :], w, preferred_element_type=jnp.float32) + b
        out_ref[t] = g.astype(out_ref.dtype)


def _scan_kernel(gih_ref,
                 whh0_ref,
                 wih1_ref,
                 whh1_ref,
                 b1_ref,
                 wfc_ref,
                 bfc_ref,
                 out_ref,
                 h1_ref, c0_ref, c1_ref,
                 m0_ref):
    chunk = pl.program_id(1)
    Bh, H = h1_ref.shape
    four_h = 4 * H
    tc = gih_ref.shape[0]

    @pl.when(chunk == 0)
    def _():
        h1_ref[...] = jnp.zeros_like(h1_ref)
        c0_ref[...] = jnp.zeros_like(c0_ref)
        c1_ref[...] = jnp.zeros_like(c1_ref)
        m0_ref[...] = jnp.zeros_like(m0_ref)

    whh0 = whh0_ref[...]
    wih1 = wih1_ref[...]
    whh1 = whh1_ref[...]
    b1 = jnp.broadcast_to(b1_ref[...], (Bh, four_h))

    def act(g):
        i = jax.nn.sigmoid(g[:, 0 * H:1 * H])
        f = jax.nn.sigmoid(g[:, 1 * H:2 * H])
        gg = jnp.tanh(g[:, 2 * H:3 * H])
        o = jax.nn.sigmoid(g[:, 3 * H:4 * H])
        return i, f, gg, o

    def step(t, carry):
        h1, c0, c1, m0 = carry

        g0 = gih_ref[t].astype(jnp.float32) + m0
        i0, f0, g0g, o0 = act(g0)
        c0n = f0 * c0 + i0 * g0g
        h0n = o0 * jnp.tanh(c0n)

        m0n = jnp.dot(h0n, whh0, preferred_element_type=jnp.float32)
        g1 = (jnp.dot(h0n, wih1, preferred_element_type=jnp.float32)
              + jnp.dot(h1, whh1, preferred_element_type=jnp.float32) + b1)
        i1, f1, g1g, o1 = act(g1)
        c1n = f1 * c1 + i1 * g1g
        h1n = o1 * jnp.tanh(c1n)

        return h1n, c0n, c1n, m0n

    carry = (h1_ref[...], c0_ref[...], c1_ref[...], m0_ref[...])
    carry = jax.lax.fori_loop(0, tc, step, carry, unroll=4)
    h1n, c0n, c1n, m0n = carry

    h1_ref[...] = h1n
    c0_ref[...] = c0n
    c1_ref[...] = c1n
    m0_ref[...] = m0n

    @pl.when(chunk == pl.num_programs(1) - 1)
    def _():
        out_ref[...] = (jnp.dot(h1n, wfc_ref[...], preferred_element_type=jnp.float32)
                        + bfc_ref[...])


def _pick_chunk(T, target):
    best = None
    for tc in range(1, T + 1):
        if T % tc == 0 and tc <= target and (tc % 8 == 0 or best is None):
            best = tc
    return best if best is not None else T


def kernel(x, wih0, whh0, b0, wih1, whh1, b1, wfc, bfc):
    B, T, F = x.shape
    H = whh0.shape[0]
    four_h = 4 * H
    O = wfc.shape[1]

    tcp = _pick_chunk(T, 40)
    pp = T // tcp
    gih = pl.pallas_call(
        _proj_kernel,
        out_shape=jax.ShapeDtypeStruct((T, B, four_h), jnp.bfloat16),
        grid=(pp,),
        in_specs=[
            pl.BlockSpec((B, tcp, F), lambda p: (0, p, 0)),
            pl.BlockSpec((F, four_h), lambda p: (0, 0)),
            pl.BlockSpec((1, four_h), lambda p: (0, 0)),
        ],
        out_specs=pl.BlockSpec((tcp, B, four_h), lambda p: (p, 0, 0)),
        compiler_params=pltpu.CompilerParams(
            dimension_semantics=("parallel",)),
    )(x, wih0, b0)

    nb = 2 if (B % 2 == 0 and (B // 2) % 8 == 0) else 1
    Bh = B // nb
    tc = _pick_chunk(T, 40)
    nc = T // tc

    out = pl.pallas_call(
        _scan_kernel,
        out_shape=jax.ShapeDtypeStruct((B, O), jnp.float32),
        grid=(nb, nc),
        in_specs=[
            pl.BlockSpec((tc, Bh, four_h), lambda b, c: (c, b, 0)),
            pl.BlockSpec((H, four_h), lambda b, c: (0, 0)),
            pl.BlockSpec((H, four_h), lambda b, c: (0, 0)),
            pl.BlockSpec((H, four_h), lambda b, c: (0, 0)),
            pl.BlockSpec((1, four_h), lambda b, c: (0, 0)),
            pl.BlockSpec((H, O), lambda b, c: (0, 0)),
            pl.BlockSpec((1, O), lambda b, c: (0, 0)),
        ],
        out_specs=pl.BlockSpec((Bh, O), lambda b, c: (b, 0)),
        scratch_shapes=[
            pltpu.VMEM((Bh, H), jnp.float32),
            pltpu.VMEM((Bh, H), jnp.float32),
            pltpu.VMEM((Bh, H), jnp.float32),
            pltpu.VMEM((Bh, four_h), jnp.float32),
        ],
        compiler_params=pltpu.CompilerParams(
            dimension_semantics=("parallel", "arbitrary")),
    )(gih, whh0, wih1, whh1, b1, wfc, bfc)

    return out[:, None, :]
